# baseline (speedup 1.0000x reference)
import jax
import jax.numpy as jnp
from jax import lax
from jax.experimental import pallas as pl
from jax.experimental.pallas import tpu as pltpu
from jax.experimental.pallas import tpu_sc as plsc

_BATCH = 16384
_HID = 16
_NW = 32
_PER_W = _BATCH // _NW
_IDX_CHUNK = 128
_N_CHUNK = _PER_W // _IDX_CHUNK


def _mf_body(u_idx_hbm, i_idx_hbm, user_w_hbm, item_w_hbm, out_hbm,
             uidx_v, iidx_v, urows_v, irows_v, out_v, sem):
    nc = 2
    wid = lax.axis_index("s") * nc + lax.axis_index("c")
    base = wid * _PER_W

    pltpu.sync_copy(u_idx_hbm.at[wid], uidx_v)
    pltpu.sync_copy(i_idx_hbm.at[wid], iidx_v)

    copies = []
    for j in range(_N_CHUNK):
        sl = pl.ds(j * _IDX_CHUNK, _IDX_CHUNK)
        copies.append(pltpu.async_copy(
            user_w_hbm.at[uidx_v.at[j]], urows_v.at[sl], sem))
        copies.append(pltpu.async_copy(
            item_w_hbm.at[iidx_v.at[j]], irows_v.at[sl], sem))
    for c in copies:
        c.wait()

    lane = lax.iota(jnp.int32, 16)

    def chunk(c, carry):
        bvec = lane + c * 16
        acc = jnp.zeros((16,), jnp.float32)
        for d in range(_HID):
            dvec = jnp.full((16,), d, jnp.int32)
            pv = plsc.load_gather(urows_v, [bvec, dvec])
            qv = plsc.load_gather(irows_v, [bvec, dvec])
            acc = acc + pv * qv
        out_v[pl.ds(c * 16, 16)] = acc
        return carry

    lax.fori_loop(0, _PER_W // 16, chunk, 0)

    pltpu.sync_copy(out_v, out_hbm.at[pl.ds(base, _PER_W)])


def kernel(user_indices, item_indices, embed_user_w, embed_item_w):
    u_idx = user_indices.astype(jnp.int32).reshape(_NW, _N_CHUNK, _IDX_CHUNK)
    i_idx = item_indices.astype(jnp.int32).reshape(_NW, _N_CHUNK, _IDX_CHUNK)

    mesh = plsc.VectorSubcoreMesh(core_axis_name="c", subcore_axis_name="s")
    run = pl.kernel(
        _mf_body, mesh=mesh,
        out_type=jax.ShapeDtypeStruct((_BATCH,), jnp.float32),
        scratch_types=[
            pltpu.VMEM((_N_CHUNK, _IDX_CHUNK), jnp.int32),
            pltpu.VMEM((_N_CHUNK, _IDX_CHUNK), jnp.int32),
            pltpu.VMEM((_PER_W, _HID), jnp.float32),
            pltpu.VMEM((_PER_W, _HID), jnp.float32),
            pltpu.VMEM((_PER_W,), jnp.float32),
            pltpu.SemaphoreType.DMA,
        ],
        compiler_params=pltpu.CompilerParams(
            needs_layout_passes=False, use_tc_tiling_on_sc=False),
    )
    return run(u_idx, i_idx, embed_user_w, embed_item_w)


# speedup vs baseline: 5.6661x; 5.6661x over previous
import jax
import jax.numpy as jnp
from jax import lax
from jax.experimental import pallas as pl
from jax.experimental.pallas import tpu as pltpu
from jax.experimental.pallas import tpu_sc as plsc

_BATCH = 16384
_HID = 16
_NW = 32
_PER_W = _BATCH // _NW
_CH = 16


def _mf_body(u_idx_hbm, i_idx_hbm, u_t3_hbm, i_t3_hbm, out_hbm,
             uidx_v, iidx_v, ub_v, ib_v, prods_v, out_v, sem):
    nc = 2
    wid = lax.axis_index("s") * nc + lax.axis_index("c")

    pltpu.sync_copy(u_idx_hbm.at[wid], uidx_v)
    pltpu.sync_copy(i_idx_hbm.at[wid], iidx_v)

    lane = lax.iota(jnp.int32, 16)
    i_vec = lane // 8
    d_vec = lane % 8
    base_pat = i_vec * 1024 + d_vec * 128

    def chunk(c, carry):
        iu = uidx_v[0, pl.ds(c * _CH, _CH)]
        ii = iidx_v[0, pl.ds(c * _CH, _CH)]
        copies = []
        for k in range(_CH):
            ru = iu[k]
            ri = ii[k]
            bu = pl.multiple_of((ru // 128) * 128, 128)
            bi = pl.multiple_of((ri // 128) * 128, 128)
            copies.append(pltpu.async_copy(
                u_t3_hbm.at[:, :, pl.ds(bu, 128)], ub_v.at[k], sem))
            copies.append(pltpu.async_copy(
                i_t3_hbm.at[:, :, pl.ds(bi, 128)], ib_v.at[k], sem))
        for cp in copies:
            cp.wait()
        for k in range(_CH):
            cu = jnp.full((16,), iu[k] % 128, jnp.int32)
            ci = jnp.full((16,), ii[k] % 128, jnp.int32)
            kk = jnp.full((16,), k, jnp.int32)
            uv = plsc.load_gather(ub_v, [kk, i_vec, d_vec, cu])
            iv = plsc.load_gather(ib_v, [kk, i_vec, d_vec, ci])
            prods_v[k, pl.ds(0, 16)] = uv * iv
        acc = jnp.zeros((16,), jnp.float32)
        for d in range(_HID):
            dd = jnp.full((16,), d, jnp.int32)
            acc = acc + plsc.load_gather(prods_v, [lane, dd])
        out_v[0, pl.ds(c * _CH, _CH)] = acc
        return carry

    lax.fori_loop(0, _PER_W // _CH, chunk, 0)

    pltpu.sync_copy(out_v, out_hbm.at[wid])


def kernel(user_indices, item_indices, embed_user_w, embed_item_w):
    u_idx = user_indices.astype(jnp.int32).reshape(_NW, 1, _PER_W)
    i_idx = item_indices.astype(jnp.int32).reshape(_NW, 1, _PER_W)
    u_t3 = embed_user_w.T.reshape(2, 8, 1000001)
    i_t3 = embed_item_w.T.reshape(2, 8, 1000001)

    mesh = plsc.VectorSubcoreMesh(core_axis_name="c", subcore_axis_name="s")
    run = pl.kernel(
        _mf_body, mesh=mesh,
        out_type=jax.ShapeDtypeStruct((_NW, 1, _PER_W), jnp.float32),
        scratch_types=[
            pltpu.VMEM((1, _PER_W), jnp.int32),
            pltpu.VMEM((1, _PER_W), jnp.int32),
            pltpu.VMEM((_CH, 2, 8, 128), jnp.float32),
            pltpu.VMEM((_CH, 2, 8, 128), jnp.float32),
            pltpu.VMEM((_CH, 128), jnp.float32),
            pltpu.VMEM((1, _PER_W), jnp.float32),
            pltpu.SemaphoreType.DMA,
        ],
        compiler_params=pltpu.CompilerParams(needs_layout_passes=False),
    )
    out = run(u_idx, i_idx, u_t3, i_t3)
    return out.reshape(_BATCH)


# speedup vs baseline: 5.9686x; 1.0534x over previous
import jax
import jax.numpy as jnp
from jax import lax
from jax.experimental import pallas as pl
from jax.experimental.pallas import tpu as pltpu
from jax.experimental.pallas import tpu_sc as plsc

_BATCH = 16384
_HID = 16
_NW = 32
_PER_W = _BATCH // _NW
_CH = 8
_NCHUNK = _PER_W // _CH


def _mf_body(u_idx_hbm, i_idx_hbm, u_t3_hbm, i_t3_hbm, out_hbm,
             uidx_v, iidx_v, ub_a, ib_a, ub_b, ib_b, prods_v, out_v,
             sem_a, sem_b):
    nc = 2
    wid = lax.axis_index("s") * nc + lax.axis_index("c")

    pltpu.sync_copy(u_idx_hbm.at[wid], uidx_v)
    pltpu.sync_copy(i_idx_hbm.at[wid], iidx_v)

    lane = lax.iota(jnp.int32, 16)
    i_vec = lane // 8
    d_vec = lane % 8

    def fire(j, half, ub, ib, sem):
        iu = uidx_v[0, pl.ds(j * 16, 16)]
        ii = iidx_v[0, pl.ds(j * 16, 16)]
        for k in range(_CH):
            bu = pl.multiple_of((iu[half * _CH + k] // 128) * 128, 128)
            bi = pl.multiple_of((ii[half * _CH + k] // 128) * 128, 128)
            pltpu.async_copy(u_t3_hbm.at[:, :, pl.ds(bu, 128)], ub.at[k], sem)
            pltpu.async_copy(i_t3_hbm.at[:, :, pl.ds(bi, 128)], ib.at[k], sem)

    def drain(ub, ib, sem):
        dummy = u_t3_hbm.at[:, :, pl.ds(0, 128)]
        for k in range(_CH):
            pltpu.make_async_copy(dummy, ub.at[k], sem).wait()
            pltpu.make_async_copy(dummy, ib.at[k], sem).wait()

    def compute_half(j, ub, ib, half):
        iu = uidx_v[0, pl.ds(j * 16, 16)]
        ii = iidx_v[0, pl.ds(j * 16, 16)]
        for k in range(_CH):
            cu = jnp.full((16,), iu[half * _CH + k] % 128, jnp.int32)
            ci = jnp.full((16,), ii[half * _CH + k] % 128, jnp.int32)
            kk = jnp.full((16,), k, jnp.int32)
            uv = plsc.load_gather(ub, [kk, i_vec, d_vec, cu])
            iv = plsc.load_gather(ib, [kk, i_vec, d_vec, ci])
            prods_v[half * _CH + k, pl.ds(0, 16)] = uv * iv

    fire(0, 0, ub_a, ib_a, sem_a)

    def body(j, carry):
        fire(j, 1, ub_b, ib_b, sem_b)
        drain(ub_a, ib_a, sem_a)
        compute_half(j, ub_a, ib_a, 0)

        @pl.when(j + 1 < _NCHUNK // 2)
        def _():
            fire(j + 1, 0, ub_a, ib_a, sem_a)

        drain(ub_b, ib_b, sem_b)
        compute_half(j, ub_b, ib_b, 1)

        acc = jnp.zeros((16,), jnp.float32)
        for d in range(_HID):
            dd = jnp.full((16,), d, jnp.int32)
            acc = acc + plsc.load_gather(prods_v, [lane, dd])
        out_v[0, pl.ds(j * 16, 16)] = acc
        return carry

    lax.fori_loop(0, _NCHUNK // 2, body, 0)

    pltpu.sync_copy(out_v, out_hbm.at[wid])


def kernel(user_indices, item_indices, embed_user_w, embed_item_w):
    u_idx = user_indices.astype(jnp.int32).reshape(_NW, 1, _PER_W)
    i_idx = item_indices.astype(jnp.int32).reshape(_NW, 1, _PER_W)
    u_t3 = embed_user_w.T.reshape(2, 8, 1000001)
    i_t3 = embed_item_w.T.reshape(2, 8, 1000001)

    mesh = plsc.VectorSubcoreMesh(core_axis_name="c", subcore_axis_name="s")
    run = pl.kernel(
        _mf_body, mesh=mesh,
        out_type=jax.ShapeDtypeStruct((_NW, 1, _PER_W), jnp.float32),
        scratch_types=[
            pltpu.VMEM((1, _PER_W), jnp.int32),
            pltpu.VMEM((1, _PER_W), jnp.int32),
            pltpu.VMEM((_CH, 2, 8, 128), jnp.float32),
            pltpu.VMEM((_CH, 2, 8, 128), jnp.float32),
            pltpu.VMEM((_CH, 2, 8, 128), jnp.float32),
            pltpu.VMEM((_CH, 2, 8, 128), jnp.float32),
            pltpu.VMEM((2 * _CH, 128), jnp.float32),
            pltpu.VMEM((1, _PER_W), jnp.float32),
            pltpu.SemaphoreType.DMA,
            pltpu.SemaphoreType.DMA,
        ],
        compiler_params=pltpu.CompilerParams(needs_layout_passes=False),
    )
    out = run(u_idx, i_idx, u_t3, i_t3)
    return out.reshape(_BATCH)
